# baseline (speedup 1.0000x reference)
import functools

import jax
import jax.numpy as jnp
from jax.experimental import pallas as pl

H = 768
D_FF = 3072
S = 2048
NR, NS, TOPK = 4, 2, 2
NE = NR + NS
FBLK = 768
NF = D_FF // FBLK
NEG = -1e30


def _gelu(x):
    return 0.5 * x * (1.0 + jax.lax.erf(x * 0.7071067811865476))


def _router_body(x_ref, w1_ref, b1_ref, w2_ref, b2_ref, wmat_ref, z_ref):
    x = x_ref[...]
    hr = _gelu(jnp.dot(x, w1_ref[...], preferred_element_type=jnp.float32)
               + b1_ref[...])
    logits = (jnp.dot(hr, w2_ref[...], preferred_element_type=jnp.float32)
              + b2_ref[...])
    col = jax.lax.broadcasted_iota(jnp.int32, logits.shape, 1)
    logits = jnp.where(col < NR, logits, NEG)
    m1 = jnp.max(logits, axis=-1, keepdims=True)
    idx1 = jnp.min(jnp.where(logits == m1, col, 1000), axis=-1, keepdims=True)
    l2 = jnp.where(col == idx1, NEG, logits)
    m2 = jnp.max(l2, axis=-1, keepdims=True)
    idx2 = jnp.min(jnp.where(l2 == m2, col, 1000), axis=-1, keepdims=True)
    e2 = jnp.exp(m2 - m1)
    wa = 1.0 / (1.0 + e2)
    wb = e2 * wa
    wmat = jnp.where(col == idx1, wa, 0.0) + jnp.where(col == idx2, wb, 0.0)
    wmat = jnp.where(col < NR, wmat,
                     jnp.where(col < NR + NS, 1.0 / NS, 0.0))
    wmat_ref[...] = wmat
    lse = m1[:, 0] + jnp.log(jnp.sum(jnp.exp(logits - m1), axis=-1))
    z_ref[...] = jnp.mean(jnp.square(lse)).reshape(1, 1)


def _expert_body(x_ref, rw1_ref, rb1_ref, rw2_ref, rb2_ref,
                 sw1_ref, sb1_ref, sw2_ref, sb2_ref, wmat_ref, acc_ref):
    e = pl.program_id(0)
    f = pl.program_id(1)

    @pl.when(jnp.logical_and(e == 0, f == 0))
    def _init():
        acc_ref[...] = jnp.zeros_like(acc_ref)

    wcol = jnp.zeros((S, 1), jnp.float32)
    for j in range(NE):
        wcol = wcol + jnp.where(e == j, wmat_ref[:, j:j + 1], 0.0)
    x = x_ref[...]

    def ffn_step(w1_ref_, b1_ref_, w2_ref_, b2_ref_):
        w1 = w1_ref_[0].astype(jnp.bfloat16)
        w2 = w2_ref_[0].astype(jnp.bfloat16)
        h = _gelu(jnp.dot(x, w1, preferred_element_type=jnp.float32)
                  + b1_ref_[0, 0])
        contrib = jnp.dot(h.astype(jnp.bfloat16), w2,
                          preferred_element_type=jnp.float32)

        @pl.when(f == 0)
        def _bias():
            acc_ref[...] += wcol * b2_ref_[0, 0][None, :]

        acc_ref[...] += wcol * contrib

    @pl.when(e < NR)
    def _routed():
        ffn_step(rw1_ref, rb1_ref, rw2_ref, rb2_ref)

    @pl.when(e >= NR)
    def _shared():
        ffn_step(sw1_ref, sb1_ref, sw2_ref, sb2_ref)


def _finish_body(acc_ref, x_ref, w_ref, b_ref, out_ref):
    o = jnp.dot(acc_ref[...], w_ref[...], preferred_element_type=jnp.float32)
    o = (o + b_ref[...]) * 0.5 + x_ref[...] * 0.5
    mean = jnp.mean(o, axis=-1, keepdims=True)
    o = o - mean
    var = jnp.mean(jnp.square(o), axis=-1, keepdims=True)
    out_ref[...] = o * jax.lax.rsqrt(var + 1e-6)


def _const_spec(shape):
    return pl.BlockSpec(shape, lambda *_: tuple(0 for _ in shape))


@functools.partial(jax.jit, static_argnames=("interpret",))
def _run(x2d, router_w1, router_b1, router_w2p, router_b2p,
         re_w1, re_b1, re_w2, re_b2,
         se_w1, se_b1, se_w2, se_b2, out_w, out_b, interpret=False):
    wmat, z_loss = pl.pallas_call(
        _router_body,
        grid=(1,),
        in_specs=[_const_spec((S, H)), _const_spec((H, H)),
                  _const_spec((1, H)), _const_spec((H, 8)),
                  _const_spec((1, 8))],
        out_specs=[_const_spec((S, 8)), _const_spec((1, 1))],
        out_shape=[jax.ShapeDtypeStruct((S, 8), jnp.float32),
                   jax.ShapeDtypeStruct((1, 1), jnp.float32)],
        interpret=interpret,
    )(x2d, router_w1, router_b1.reshape(1, H), router_w2p, router_b2p)
    x_bf = x2d.astype(jnp.bfloat16)

    def re_w1_idx(e, f):
        return (jnp.minimum(e, NR - 1), 0, jnp.where(e < NR, f, NF - 1))

    def se_w1_idx(e, f):
        return (jnp.clip(e - NR, 0, NS - 1), 0, jnp.where(e < NR, 0, f))

    def re_w2_idx(e, f):
        return (jnp.minimum(e, NR - 1), jnp.where(e < NR, f, NF - 1), 0)

    def se_w2_idx(e, f):
        return (jnp.clip(e - NR, 0, NS - 1), jnp.where(e < NR, 0, f), 0)

    acc = pl.pallas_call(
        _expert_body,
        grid=(NE, NF),
        in_specs=[
            _const_spec((S, H)),
            pl.BlockSpec((1, H, FBLK), re_w1_idx),
            pl.BlockSpec((1, 1, FBLK),
                         lambda e, f: (jnp.minimum(e, NR - 1), 0,
                                       jnp.where(e < NR, f, NF - 1))),
            pl.BlockSpec((1, FBLK, H), re_w2_idx),
            pl.BlockSpec((1, 1, H), lambda e, f: (jnp.minimum(e, NR - 1), 0, 0)),
            pl.BlockSpec((1, H, FBLK), se_w1_idx),
            pl.BlockSpec((1, 1, FBLK),
                         lambda e, f: (jnp.clip(e - NR, 0, NS - 1), 0,
                                       jnp.where(e < NR, 0, f))),
            pl.BlockSpec((1, FBLK, H), se_w2_idx),
            pl.BlockSpec((1, 1, H), lambda e, f: (jnp.clip(e - NR, 0, NS - 1), 0, 0)),
            _const_spec((S, 8)),
        ],
        out_specs=_const_spec((S, H)),
        out_shape=jax.ShapeDtypeStruct((S, H), jnp.float32),
        interpret=interpret,
    )(x_bf, re_w1, re_b1.reshape(NR, 1, D_FF), re_w2, re_b2.reshape(NR, 1, H),
      se_w1, se_b1.reshape(NS, 1, D_FF), se_w2, se_b2.reshape(NS, 1, H), wmat)

    out = pl.pallas_call(
        _finish_body,
        grid=(1,),
        in_specs=[_const_spec((S, H)), _const_spec((S, H)),
                  _const_spec((H, H)), _const_spec((1, H))],
        out_specs=_const_spec((S, H)),
        out_shape=jax.ShapeDtypeStruct((S, H), jnp.float32),
        interpret=interpret,
    )(acc, x2d, out_w, out_b.reshape(1, H))
    return out, z_loss


def kernel(hidden_states, router_w1, router_b1, router_w2, router_b2,
           re_w1, re_b1, re_w2, re_b2, se_w1, se_b1, se_w2, se_b2,
           out_w, out_b, interpret=False):
    x2d = hidden_states.reshape(S, H)
    router_w2p = jnp.pad(router_w2, ((0, 0), (0, 8 - NR)))
    router_b2p = jnp.pad(router_b2, (0, 8 - NR)).reshape(1, 8)
    out, z_loss = _run(x2d, router_w1, router_b1, router_w2p, router_b2p,
                       re_w1, re_b1, re_w2, re_b2,
                       se_w1, se_b1, se_w2, se_b2, out_w, out_b,
                       interpret=interpret)
    return out.reshape(1, S, H), z_loss[0, 0]


# speedup vs baseline: 1.1967x; 1.1967x over previous
import functools

import jax
import jax.numpy as jnp
from jax.experimental import pallas as pl

H = 768
D_FF = 3072
S = 2048
NR, NS, TOPK = 4, 2, 2
NE = NR + NS
FBLK = 768
NF = D_FF // FBLK
NEG = -1e30


def _gelu(x):
    return 0.5 * x * (1.0 + jax.lax.erf(x * 0.7071067811865476))


def _router_body(x_ref, w1_ref, b1_ref, w2_ref, b2_ref, wmat_ref, z_ref):
    x = x_ref[...]
    hr = _gelu(jnp.dot(x, w1_ref[...], preferred_element_type=jnp.float32)
               + b1_ref[...])
    logits = (jnp.dot(hr, w2_ref[...], preferred_element_type=jnp.float32)
              + b2_ref[...])
    col = jax.lax.broadcasted_iota(jnp.int32, logits.shape, 1)
    logits = jnp.where(col < NR, logits, NEG)
    m1 = jnp.max(logits, axis=-1, keepdims=True)
    idx1 = jnp.min(jnp.where(logits == m1, col, 1000), axis=-1, keepdims=True)
    l2 = jnp.where(col == idx1, NEG, logits)
    m2 = jnp.max(l2, axis=-1, keepdims=True)
    idx2 = jnp.min(jnp.where(l2 == m2, col, 1000), axis=-1, keepdims=True)
    e2 = jnp.exp(m2 - m1)
    wa = 1.0 / (1.0 + e2)
    wb = e2 * wa
    wmat = jnp.where(col == idx1, wa, 0.0) + jnp.where(col == idx2, wb, 0.0)
    wmat = jnp.where(col < NR, wmat,
                     jnp.where(col < NR + NS, 1.0 / NS, 0.0))
    wmat_ref[...] = wmat
    lse = m1[:, 0] + jnp.log(jnp.sum(jnp.exp(logits - m1), axis=-1))
    z_ref[...] = jnp.mean(jnp.square(lse)).reshape(1, 1)


def _expert_body(x_ref, rw1_ref, rb1_ref, rw2_ref, rb2_ref,
                 sw1_ref, sb1_ref, sw2_ref, sb2_ref, wmat_ref, acc_ref):
    e = pl.program_id(0)
    f = pl.program_id(1)

    @pl.when(jnp.logical_and(e == 0, f == 0))
    def _init():
        acc_ref[...] = jnp.zeros_like(acc_ref)

    wcol = jnp.zeros((S, 1), jnp.float32)
    for j in range(NE):
        wcol = wcol + jnp.where(e == j, wmat_ref[:, j:j + 1], 0.0)
    x = x_ref[...]

    routed = e < NR
    w1 = jnp.where(routed, rw1_ref[0], sw1_ref[0])
    w2 = jnp.where(routed, rw2_ref[0], sw2_ref[0])
    b1 = jnp.where(routed, rb1_ref[0, 0], sb1_ref[0, 0])
    h = _gelu(jnp.dot(x, w1, preferred_element_type=jnp.float32) + b1)
    contrib = jnp.dot(h, w2, preferred_element_type=jnp.float32)

    @pl.when(f == 0)
    def _bias():
        b2 = jnp.where(routed, rb2_ref[0, 0], sb2_ref[0, 0])
        acc_ref[...] += wcol * b2[None, :]

    acc_ref[...] += wcol * contrib


def _finish_body(acc_ref, x_ref, w_ref, b_ref, out_ref):
    o = jnp.dot(acc_ref[...], w_ref[...], preferred_element_type=jnp.float32)
    o = (o + b_ref[...]) * 0.5 + x_ref[...] * 0.5
    mean = jnp.mean(o, axis=-1, keepdims=True)
    o = o - mean
    var = jnp.mean(jnp.square(o), axis=-1, keepdims=True)
    out_ref[...] = o * jax.lax.rsqrt(var + 1e-6)


def _const_spec(shape):
    return pl.BlockSpec(shape, lambda *_: tuple(0 for _ in shape))


@functools.partial(jax.jit, static_argnames=("interpret",))
def _run(x2d, router_w1, router_b1, router_w2p, router_b2p,
         re_w1, re_b1, re_w2, re_b2,
         se_w1, se_b1, se_w2, se_b2, out_w, out_b, interpret=False):
    wmat, z_loss = pl.pallas_call(
        _router_body,
        grid=(1,),
        in_specs=[_const_spec((S, H)), _const_spec((H, H)),
                  _const_spec((1, H)), _const_spec((H, 8)),
                  _const_spec((1, 8))],
        out_specs=[_const_spec((S, 8)), _const_spec((1, 1))],
        out_shape=[jax.ShapeDtypeStruct((S, 8), jnp.float32),
                   jax.ShapeDtypeStruct((1, 1), jnp.float32)],
        interpret=interpret,
    )(x2d, router_w1, router_b1.reshape(1, H), router_w2p, router_b2p)

    def re_w1_idx(e, f):
        return (jnp.minimum(e, NR - 1), 0, jnp.where(e < NR, f, NF - 1))

    def se_w1_idx(e, f):
        return (jnp.clip(e - NR, 0, NS - 1), 0, jnp.where(e < NR, 0, f))

    def re_w2_idx(e, f):
        return (jnp.minimum(e, NR - 1), jnp.where(e < NR, f, NF - 1), 0)

    def se_w2_idx(e, f):
        return (jnp.clip(e - NR, 0, NS - 1), jnp.where(e < NR, 0, f), 0)

    acc = pl.pallas_call(
        _expert_body,
        grid=(NE, NF),
        in_specs=[
            _const_spec((S, H)),
            pl.BlockSpec((1, H, FBLK), re_w1_idx),
            pl.BlockSpec((1, 1, FBLK),
                         lambda e, f: (jnp.minimum(e, NR - 1), 0,
                                       jnp.where(e < NR, f, NF - 1))),
            pl.BlockSpec((1, FBLK, H), re_w2_idx),
            pl.BlockSpec((1, 1, H), lambda e, f: (jnp.minimum(e, NR - 1), 0, 0)),
            pl.BlockSpec((1, H, FBLK), se_w1_idx),
            pl.BlockSpec((1, 1, FBLK),
                         lambda e, f: (jnp.clip(e - NR, 0, NS - 1), 0,
                                       jnp.where(e < NR, 0, f))),
            pl.BlockSpec((1, FBLK, H), se_w2_idx),
            pl.BlockSpec((1, 1, H), lambda e, f: (jnp.clip(e - NR, 0, NS - 1), 0, 0)),
            _const_spec((S, 8)),
        ],
        out_specs=_const_spec((S, H)),
        out_shape=jax.ShapeDtypeStruct((S, H), jnp.float32),
        interpret=interpret,
    )(x2d, re_w1, re_b1.reshape(NR, 1, D_FF), re_w2, re_b2.reshape(NR, 1, H),
      se_w1, se_b1.reshape(NS, 1, D_FF), se_w2, se_b2.reshape(NS, 1, H), wmat)

    out = pl.pallas_call(
        _finish_body,
        grid=(1,),
        in_specs=[_const_spec((S, H)), _const_spec((S, H)),
                  _const_spec((H, H)), _const_spec((1, H))],
        out_specs=_const_spec((S, H)),
        out_shape=jax.ShapeDtypeStruct((S, H), jnp.float32),
        interpret=interpret,
    )(acc, x2d, out_w, out_b.reshape(1, H))
    return out, z_loss


def kernel(hidden_states, router_w1, router_b1, router_w2, router_b2,
           re_w1, re_b1, re_w2, re_b2, se_w1, se_b1, se_w2, se_b2,
           out_w, out_b, interpret=False):
    x2d = hidden_states.reshape(S, H)
    router_w2p = jnp.pad(router_w2, ((0, 0), (0, 8 - NR)))
    router_b2p = jnp.pad(router_b2, (0, 8 - NR)).reshape(1, 8)
    out, z_loss = _run(x2d, router_w1, router_b1, router_w2p, router_b2p,
                       re_w1, re_b1, re_w2, re_b2,
                       se_w1, se_b1, se_w2, se_b2, out_w, out_b,
                       interpret=interpret)
    return out.reshape(1, S, H), z_loss[0, 0]


# speedup vs baseline: 1.2252x; 1.0238x over previous
import functools

import jax
import jax.numpy as jnp
from jax import lax
from jax.experimental import pallas as pl
from jax.experimental.pallas import tpu as pltpu
from jax.experimental.pallas import tpu_sc as plsc

H = 768
D_FF = 3072
S = 2048
NR, NS = 4, 2
FBLK = 1536
NF = D_FF // FBLK
NEG = -1e30

NWORKERS = 32
TOKW = S // NWORKERS
LANES = 16


def _gelu(x):
    return 0.5 * x * (1.0 + jax.lax.erf(x * 0.7071067811865476))


def _router_core(x_ref, w1_ref, b1_ref, w2t_ref, b2t_ref, logits_ref, z_ref):
    x = x_ref[...]
    hr = _gelu(jnp.dot(x, w1_ref[...], preferred_element_type=jnp.float32)
               + b1_ref[...])
    logits = lax.dot_general(w2t_ref[...], hr, (((1,), (1,)), ((), ())),
                             preferred_element_type=jnp.float32) + b2t_ref[...]
    row = jax.lax.broadcasted_iota(jnp.int32, logits.shape, 0)
    logits = jnp.where(row < NR, logits, NEG)
    logits_ref[...] = logits
    m1 = jnp.max(logits, axis=0)
    lse = m1 + jnp.log(jnp.sum(jnp.exp(logits - m1[None, :]), axis=0))
    z_ref[...] = jnp.mean(jnp.square(lse)).reshape(1, 1)


def _sc_topk_body(logits_hbm, wmat_hbm, lg_v, wm_v):
    wid = lax.axis_index("s") * 2 + lax.axis_index("c")
    base = wid * TOKW
    for j in range(NR):
        pltpu.sync_copy(logits_hbm.at[pl.ds(j * S + base, TOKW)], lg_v.at[j])
    for i in range(TOKW // LANES):
        lsl = pl.ds(LANES * i, LANES)
        l = [lg_v[j, lsl] for j in range(NR)]
        m1 = jnp.maximum(jnp.maximum(l[0], l[1]), jnp.maximum(l[2], l[3]))
        idx1 = jnp.where(l[0] == m1, 0,
                         jnp.where(l[1] == m1, 1,
                                   jnp.where(l[2] == m1, 2, 3)))
        l2 = [jnp.where(idx1 == j, NEG, l[j]) for j in range(NR)]
        m2 = jnp.maximum(jnp.maximum(l2[0], l2[1]),
                         jnp.maximum(l2[2], l2[3]))
        idx2 = jnp.where(l2[0] == m2, 0,
                         jnp.where(l2[1] == m2, 1,
                                   jnp.where(l2[2] == m2, 2, 3)))
        e2 = jnp.exp(m2 - m1)
        wa = 1.0 / (1.0 + e2)
        wb = e2 * wa
        sl = pl.ds(LANES * i, LANES)
        for j in range(NR):
            wm_v[j, sl] = (jnp.where(idx1 == j, wa, 0.0)
                           + jnp.where(idx2 == j, wb, 0.0))
        half = jnp.full((LANES,), 1.0 / NS, jnp.float32)
        zero = jnp.zeros((LANES,), jnp.float32)
        for j in range(NR, NR + NS):
            wm_v[j, sl] = half
        for j in range(NR + NS, 8):
            wm_v[j, sl] = zero
    for j in range(8):
        pltpu.sync_copy(wm_v.at[j], wmat_hbm.at[pl.ds(j * S + base, TOKW)])


def _shared_body(x_ref, rw1_ref, rb1_ref, rw2t_ref, rb2t_ref,
                 w1_ref, b1_ref, w2_ref, b2_ref,
                 acc_ref, logits_ref, z_ref):
    g = pl.program_id(0)

    @pl.when(g == 0)
    def _router():
        _router_core(x_ref, rw1_ref, rb1_ref, rw2t_ref, rb2t_ref,
                     logits_ref, z_ref)

    @pl.when(g >= 1)
    def _expert():
        f = (g - 1) % NF
        h = _gelu(jnp.dot(x_ref[...], w1_ref[0],
                          preferred_element_type=jnp.float32) + b1_ref[0, 0])
        delta = (1.0 / NS) * jnp.dot(h, w2_ref[0],
                                     preferred_element_type=jnp.float32)

        @pl.when(f == 0)
        def _bias():
            acc_ref[...] = jnp.where(g == 1, 0.0, acc_ref[...]) \
                + (1.0 / NS) * b2_ref[0, 0][None, :]

        acc_ref[...] += delta


def _routed_body(accs_ref, x_ref, wmat_ref, w1_ref, b1_ref, w2_ref, b2_ref,
                 ow_ref, ob_ref, out_ref):
    g = pl.program_id(0)

    @pl.when(g == 0)
    def _init():
        out_ref[...] = accs_ref[...]

    @pl.when(g < NR * NF)
    def _expert():
        e = g // NF
        f = g % NF
        wcol = jnp.zeros((S,), jnp.float32)
        for j in range(NR):
            wcol = wcol + jnp.where(e == j, wmat_ref[j, :], 0.0)
        h = _gelu(jnp.dot(x_ref[...], w1_ref[0],
                          preferred_element_type=jnp.float32) + b1_ref[0, 0])
        delta = wcol[:, None] * jnp.dot(h, w2_ref[0],
                                        preferred_element_type=jnp.float32)

        @pl.when(f == 0)
        def _bias():
            out_ref[...] += wcol[:, None] * b2_ref[0, 0][None, :]

        out_ref[...] += delta

    @pl.when(g == NR * NF)
    def _finish():
        o = jnp.dot(out_ref[...], ow_ref[...],
                    preferred_element_type=jnp.float32)
        o = (o + ob_ref[...]) * 0.5 + x_ref[...] * 0.5
        mean = jnp.mean(o, axis=-1, keepdims=True)
        o = o - mean
        var = jnp.mean(jnp.square(o), axis=-1, keepdims=True)
        out_ref[...] = o * jax.lax.rsqrt(var + 1e-6)


def _const_spec(shape):
    return pl.BlockSpec(shape, lambda *_: tuple(0 for _ in shape))


_sc_topk = functools.partial(
    pl.kernel,
    mesh=plsc.VectorSubcoreMesh(core_axis_name="c", subcore_axis_name="s"),
    out_type=jax.ShapeDtypeStruct((8 * S,), jnp.float32),
    scratch_types=[pltpu.VMEM((NR, TOKW), jnp.float32),
                   pltpu.VMEM((8, TOKW), jnp.float32)],
)(_sc_topk_body)


@functools.partial(jax.jit, static_argnames=("interpret",))
def _run(x2d, router_w1, router_b1, router_w2p, router_b2p,
         re_w1, re_b1, re_w2, re_b2,
         se_w1, se_b1, se_w2, se_b2, out_w, out_b, interpret=False):
    def sidx(g):
        gg = jnp.maximum(g - 1, 0)
        return gg // NF, gg % NF

    accs, logits, z_loss = pl.pallas_call(
        _shared_body,
        grid=(NS * NF + 1,),
        in_specs=[
            _const_spec((S, H)), _const_spec((H, H)),
            _const_spec((1, H)), _const_spec((8, H)), _const_spec((8, 1)),
            pl.BlockSpec((1, H, FBLK), lambda g: (sidx(g)[0], 0, sidx(g)[1])),
            pl.BlockSpec((1, 1, FBLK), lambda g: (sidx(g)[0], 0, sidx(g)[1])),
            pl.BlockSpec((1, FBLK, H), lambda g: (*sidx(g), 0)),
            pl.BlockSpec((1, 1, H), lambda g: (sidx(g)[0], 0, 0)),
        ],
        out_specs=[_const_spec((S, H)), _const_spec((8, S)),
                   _const_spec((1, 1))],
        out_shape=[jax.ShapeDtypeStruct((S, H), jnp.float32),
                   jax.ShapeDtypeStruct((8, S), jnp.float32),
                   jax.ShapeDtypeStruct((1, 1), jnp.float32)],
        interpret=interpret,
    )(x2d, router_w1, router_b1.reshape(1, H), router_w2p, router_b2p,
      se_w1, se_b1.reshape(NS, 1, D_FF), se_w2, se_b2.reshape(NS, 1, H))

    wmat = _sc_topk(logits.reshape(-1)).reshape(8, S)

    def ridx(g):
        gg = jnp.minimum(g, NR * NF - 1)
        return gg // NF, gg % NF

    out = pl.pallas_call(
        _routed_body,
        grid=(NR * NF + 1,),
        in_specs=[
            _const_spec((S, H)), _const_spec((S, H)), _const_spec((8, S)),
            pl.BlockSpec((1, H, FBLK), lambda g: (ridx(g)[0], 0, ridx(g)[1])),
            pl.BlockSpec((1, 1, FBLK), lambda g: (ridx(g)[0], 0, ridx(g)[1])),
            pl.BlockSpec((1, FBLK, H), lambda g: (*ridx(g), 0)),
            pl.BlockSpec((1, 1, H), lambda g: (ridx(g)[0], 0, 0)),
            _const_spec((H, H)), _const_spec((1, H)),
        ],
        out_specs=_const_spec((S, H)),
        out_shape=jax.ShapeDtypeStruct((S, H), jnp.float32),
        interpret=interpret,
    )(accs, x2d, wmat, re_w1, re_b1.reshape(NR, 1, D_FF),
      re_w2, re_b2.reshape(NR, 1, H), out_w, out_b.reshape(1, H))
    return out, z_loss


def kernel(hidden_states, router_w1, router_b1, router_w2, router_b2,
           re_w1, re_b1, re_w2, re_b2, se_w1, se_b1, se_w2, se_b2,
           out_w, out_b, interpret=False):
    x2d = hidden_states.reshape(S, H)
    router_w2p = jnp.pad(router_w2, ((0, 0), (0, 8 - NR))).T
    router_b2p = jnp.pad(router_b2, (0, 8 - NR)).reshape(8, 1)
    out, z_loss = _run(x2d, router_w1, router_b1, router_w2p, router_b2p,
                       re_w1, re_b1, re_w2, re_b2,
                       se_w1, se_b1, se_w2, se_b2, out_w, out_b,
                       interpret=interpret)
    return out.reshape(1, S, H), z_loss[0, 0]
